# baseline (speedup 1.0000x reference)
import functools

import jax
import jax.numpy as jnp
from jax import lax
from jax.experimental import pallas as pl
from jax.experimental.pallas import tpu as pltpu
from jax.experimental.pallas import tpu_sc as plsc

N = 10000
E = 320000
D = 128
H = 128
C = 64
K = 10
ALPHA = 0.1

NC = 2
NS = 16
LANES = 16
CHUNK = 128
NCHUNK = 80
EDGES_PER_TILE = CHUNK * NCHUNK
EPAD = NC * NS * EDGES_PER_TILE
NPAD = 10240
ROWS_PER_TILE = NPAD // NS

ROW_BLK = 2000
CMB_BLK = 2048


def _mlp_body(x_ref, w1_ref, b1_ref, w2_ref, b2_ref, h_ref, ah_ref):
    h1 = jnp.maximum(
        jnp.dot(x_ref[...], w1_ref[...], preferred_element_type=jnp.float32)
        + b1_ref[...], 0.0)
    h2 = (jnp.dot(h1, w2_ref[...], preferred_element_type=jnp.float32)
          + b2_ref[...])
    h_ref[...] = h2
    ah_ref[...] = ALPHA * h2


def _mlp(features, W1, b1, W2, b2):
    grid = (N // ROW_BLK,)
    return pl.pallas_call(
        _mlp_body,
        grid=grid,
        in_specs=[
            pl.BlockSpec((ROW_BLK, D), lambda i: (i, 0)),
            pl.BlockSpec((D, H), lambda i: (0, 0)),
            pl.BlockSpec((1, H), lambda i: (0, 0)),
            pl.BlockSpec((H, C), lambda i: (0, 0)),
            pl.BlockSpec((1, C), lambda i: (0, 0)),
        ],
        out_specs=[
            pl.BlockSpec((ROW_BLK, C), lambda i: (i, 0)),
            pl.BlockSpec((ROW_BLK, C), lambda i: (i, 0)),
        ],
        out_shape=[
            jax.ShapeDtypeStruct((N, C), jnp.float32),
            jax.ShapeDtypeStruct((N, C), jnp.float32),
        ],
    )(features, W1, b1.reshape(1, H), W2, b2.reshape(1, C))


def _combine_body(p_ref, o_ref):
    o_ref[...] = p_ref[0] + p_ref[1]


def _combine(partials):
    grid = (NPAD // CMB_BLK,)
    return pl.pallas_call(
        _combine_body,
        grid=grid,
        in_specs=[pl.BlockSpec((NC, CMB_BLK, C), lambda i: (0, i, 0))],
        out_specs=pl.BlockSpec((CMB_BLK, C), lambda i: (i, 0)),
        out_shape=jax.ShapeDtypeStruct((NPAD, C), jnp.float32),
    )(partials)


_GATHER_DNUMS = lax.GatherDimensionNumbers(
    offset_dims=(), collapsed_slice_dims=(0,), start_index_map=(0,))


def _bcast_lane(vec, lane):
    idx = jnp.full((LANES, 1), lane, jnp.int32)
    return lax.gather(vec, idx, _GATHER_DNUMS, slice_sizes=(1,),
                      mode=lax.GatherScatterMode.PROMISE_IN_BOUNDS)


NBUF = 4
SLOTS = 2 * NBUF


def _prop_body(h_hbm, pk_hbm, w_hbm, init_hbm, out_hbm,
               pk_v, w_v, idx_v, rin_v, rout_v, agg_sh, gsem, ssem):
    c = lax.axis_index("c")
    s = lax.axis_index("s")

    pltpu.sync_copy(pk_hbm.at[c, s], pk_v)
    pltpu.sync_copy(w_hbm.at[c, s], w_v)
    pltpu.sync_copy(init_hbm.at[c].at[pl.ds(s * ROWS_PER_TILE, ROWS_PER_TILE)],
                    agg_sh.at[pl.ds(s * ROWS_PER_TILE, ROWS_PER_TILE)])
    plsc.subcore_barrier()

    def unpack_idx(j, m):
        for k in range(CHUNK // LANES):
            p = pk_v[j, pl.ds(k * LANES, LANES)]
            idx_v[m, 0, pl.ds(k * LANES, LANES)] = p & 0xFFFF
            idx_v[m, 1, pl.ds(k * LANES, LANES)] = lax.shift_right_logical(p, 16)

    for b in range(NBUF):
        unpack_idx(b, b)
        pltpu.async_copy(h_hbm.at[idx_v.at[b, 0]], rin_v.at[b], gsem.at[b])

    @pl.loop(0, NCHUNK, step=NBUF)
    def _(g0):
        for b in range(NBUF):
            j = g0 + b
            msc = j % SLOTS
            pltpu.make_async_copy(h_hbm.at[idx_v.at[msc, 0]], rin_v.at[b],
                                  gsem.at[b]).wait()
            @pl.when(g0 > 0)
            def _():
                pltpu.make_async_copy(rout_v.at[b], agg_sh.at[idx_v.at[msc, 1]],
                                      ssem.at[b]).wait()
            for e5 in range(CHUNK // LANES):
                w16 = w_v[j, pl.ds(e5 * LANES, LANES)] * (1.0 - ALPHA)
                for e in range(LANES):
                    wb = _bcast_lane(w16, e)
                    row = e5 * LANES + e
                    for f in range(C // LANES):
                        sl = (row, pl.ds(f * LANES, LANES))
                        rout_v[(b,) + sl] = rin_v[(b,) + sl] * wb
            @pl.when(j + NBUF < NCHUNK)
            def _():
                mg = (j + NBUF) % SLOTS
                unpack_idx(j + NBUF, mg)
                pltpu.async_copy(h_hbm.at[idx_v.at[mg, 0]], rin_v.at[b],
                                 gsem.at[b])
            pltpu.async_copy(rout_v.at[b], agg_sh.at[idx_v.at[msc, 1]],
                             ssem.at[b], add=True)

    for b in range(NBUF):
        m = (NCHUNK - NBUF + b) % SLOTS
        pltpu.make_async_copy(rout_v.at[b], agg_sh.at[idx_v.at[m, 1]],
                              ssem.at[b]).wait()

    plsc.subcore_barrier()
    pltpu.sync_copy(agg_sh.at[pl.ds(s * ROWS_PER_TILE, ROWS_PER_TILE)],
                    out_hbm.at[c].at[pl.ds(s * ROWS_PER_TILE, ROWS_PER_TILE)])


def _make_prop():
    mesh = plsc.VectorSubcoreMesh(core_axis_name="c", subcore_axis_name="s")
    return pl.kernel(
        _prop_body,
        mesh=mesh,
        out_type=jax.ShapeDtypeStruct((NC, NPAD, C), jnp.float32),
        scratch_types=[
            pltpu.VMEM((NCHUNK, CHUNK), jnp.int32),
            pltpu.VMEM((NCHUNK, CHUNK), jnp.float32),
            pltpu.VMEM((SLOTS, 2, CHUNK), jnp.int32),
            pltpu.VMEM((NBUF, CHUNK, C), jnp.float32),
            pltpu.VMEM((NBUF, CHUNK, C), jnp.float32),
            pltpu.VMEM_SHARED((NPAD, C), jnp.float32),
            pltpu.SemaphoreType.DMA((NBUF,)),
            pltpu.SemaphoreType.DMA((NBUF,)),
        ],
        compiler_params=pltpu.CompilerParams(use_tc_tiling_on_sc=False),
    )


def kernel(features, edge_weight, edge_index, W1, b1, W2, b2):
    h0, ah0 = _mlp(features, W1, b1, W2, b2)
    epad = (0, EPAD - E)
    packed = jnp.pad(edge_index[0] + edge_index[1] * 65536,
                     epad).reshape(NC, NS, NCHUNK, CHUNK)
    w = jnp.pad(edge_weight, epad).reshape(NC, NS, NCHUNK, CHUNK)
    pad = ((0, NPAD - N), (0, 0))
    ah0p = jnp.pad(ah0, pad)
    init = jnp.stack([ah0p, jnp.zeros_like(ah0p)])
    prop = _make_prop()
    h = jnp.pad(h0, pad)
    for _ in range(K):
        partials = prop(h, packed, w, init)
        h = _combine(partials)
    return h[:N]


# speedup vs baseline: 2.9322x; 2.9322x over previous
import functools

import jax
import jax.numpy as jnp
from jax import lax
from jax.experimental import pallas as pl
from jax.experimental.pallas import tpu as pltpu
from jax.experimental.pallas import tpu_sc as plsc

N = 10000
E = 320000
D = 128
H = 128
C = 64
K = 10
ALPHA = 0.1

NC = 2
NS = 16
LANES = 16
EDGES_PER_TILE = E // (NC * NS)
CHUNK = 80
NCHUNK = EDGES_PER_TILE // CHUNK
NPAD = 10240
ROWS_PER_TILE = NPAD // NS

ROW_BLK = 2000
CMB_BLK = 2048


def _mlp_body(x_ref, w1_ref, b1_ref, w2_ref, b2_ref, h_ref, ah_ref):
    h1 = jnp.maximum(
        jnp.dot(x_ref[...], w1_ref[...], preferred_element_type=jnp.float32)
        + b1_ref[...], 0.0)
    h2 = (jnp.dot(h1, w2_ref[...], preferred_element_type=jnp.float32)
          + b2_ref[...])
    h_ref[...] = h2
    ah_ref[...] = ALPHA * h2


def _mlp(features, W1, b1, W2, b2):
    grid = (N // ROW_BLK,)
    return pl.pallas_call(
        _mlp_body,
        grid=grid,
        in_specs=[
            pl.BlockSpec((ROW_BLK, D), lambda i: (i, 0)),
            pl.BlockSpec((D, H), lambda i: (0, 0)),
            pl.BlockSpec((1, H), lambda i: (0, 0)),
            pl.BlockSpec((H, C), lambda i: (0, 0)),
            pl.BlockSpec((1, C), lambda i: (0, 0)),
        ],
        out_specs=[
            pl.BlockSpec((ROW_BLK, C), lambda i: (i, 0)),
            pl.BlockSpec((ROW_BLK, C), lambda i: (i, 0)),
        ],
        out_shape=[
            jax.ShapeDtypeStruct((N, C), jnp.float32),
            jax.ShapeDtypeStruct((N, C), jnp.float32),
        ],
    )(features, W1, b1.reshape(1, H), W2, b2.reshape(1, C))


def _combine_body(p_ref, o_ref):
    o_ref[...] = p_ref[0] + p_ref[1]


def _combine(partials):
    grid = (NPAD // CMB_BLK,)
    return pl.pallas_call(
        _combine_body,
        grid=grid,
        in_specs=[pl.BlockSpec((NC, CMB_BLK, C), lambda i: (0, i, 0))],
        out_specs=pl.BlockSpec((CMB_BLK, C), lambda i: (i, 0)),
        out_shape=jax.ShapeDtypeStruct((NPAD, C), jnp.float32),
    )(partials)


_GATHER_DNUMS = lax.GatherDimensionNumbers(
    offset_dims=(), collapsed_slice_dims=(0,), start_index_map=(0,))


def _bcast_lane(vec, lane):
    idx = jnp.full((LANES, 1), lane, jnp.int32)
    return lax.gather(vec, idx, _GATHER_DNUMS, slice_sizes=(1,),
                      mode=lax.GatherScatterMode.PROMISE_IN_BOUNDS)


NBUF = 5


def _prop_body(h_hbm, src_hbm, dst_hbm, w_hbm, init_hbm, out_hbm,
               src_v, dst_v, w_v, rin_v, rout_v, agg_sh, gsem, ssem):
    c = lax.axis_index("c")
    s = lax.axis_index("s")

    pltpu.sync_copy(src_hbm.at[c, s], src_v)
    pltpu.sync_copy(dst_hbm.at[c, s], dst_v)
    pltpu.sync_copy(w_hbm.at[c, s], w_v)
    pltpu.sync_copy(init_hbm.at[c].at[pl.ds(s * ROWS_PER_TILE, ROWS_PER_TILE)],
                    agg_sh.at[pl.ds(s * ROWS_PER_TILE, ROWS_PER_TILE)])
    plsc.subcore_barrier()

    for b in range(NBUF):
        pltpu.async_copy(h_hbm.at[src_v.at[b]], rin_v.at[b], gsem.at[b])

    @pl.loop(0, NCHUNK, step=NBUF)
    def _(g0):
        for b in range(NBUF):
            j = g0 + b
            pltpu.make_async_copy(h_hbm.at[src_v.at[j]], rin_v.at[b],
                                  gsem.at[b]).wait()
            @pl.when(g0 > 0)
            def _():
                pltpu.make_async_copy(rout_v.at[b], agg_sh.at[dst_v.at[j]],
                                      ssem.at[b]).wait()
            @pl.loop(0, CHUNK // LANES)
            def _(e5):
                w16 = w_v[j, pl.ds(e5 * LANES, LANES)] * (1.0 - ALPHA)
                for e in range(LANES):
                    wb = _bcast_lane(w16, e)
                    row = e5 * LANES + e
                    for f in range(C // LANES):
                        sl = (pl.ds(row, 1), pl.ds(f * LANES, LANES))
                        rout_v[b, row, pl.ds(f * LANES, LANES)] = (
                            rin_v[b, row, pl.ds(f * LANES, LANES)] * wb)
            @pl.when(j + NBUF < NCHUNK)
            def _():
                pltpu.async_copy(h_hbm.at[src_v.at[j + NBUF]], rin_v.at[b],
                                 gsem.at[b])
            pltpu.async_copy(rout_v.at[b], agg_sh.at[dst_v.at[j]],
                             ssem.at[b], add=True)

    for b in range(NBUF):
        pltpu.make_async_copy(rout_v.at[b],
                              agg_sh.at[dst_v.at[NCHUNK - NBUF + b]],
                              ssem.at[b]).wait()

    plsc.subcore_barrier()
    pltpu.sync_copy(agg_sh.at[pl.ds(s * ROWS_PER_TILE, ROWS_PER_TILE)],
                    out_hbm.at[c].at[pl.ds(s * ROWS_PER_TILE, ROWS_PER_TILE)])


def _make_prop():
    mesh = plsc.VectorSubcoreMesh(core_axis_name="c", subcore_axis_name="s")
    return pl.kernel(
        _prop_body,
        mesh=mesh,
        out_type=jax.ShapeDtypeStruct((NC, NPAD, C), jnp.float32),
        scratch_types=[
            pltpu.VMEM((NCHUNK, CHUNK), jnp.int32),
            pltpu.VMEM((NCHUNK, CHUNK), jnp.int32),
            pltpu.VMEM((NCHUNK, CHUNK), jnp.float32),
            pltpu.VMEM((NBUF, CHUNK, C), jnp.float32),
            pltpu.VMEM((NBUF, CHUNK, C), jnp.float32),
            pltpu.VMEM_SHARED((NPAD, C), jnp.float32),
            pltpu.SemaphoreType.DMA((NBUF,)),
            pltpu.SemaphoreType.DMA((NBUF,)),
        ],
        compiler_params=pltpu.CompilerParams(use_tc_tiling_on_sc=False),
    )


def kernel(features, edge_weight, edge_index, W1, b1, W2, b2):
    h0, ah0 = _mlp(features, W1, b1, W2, b2)
    src = edge_index[0].reshape(NC, NS, NCHUNK, CHUNK)
    dst = edge_index[1].reshape(NC, NS, NCHUNK, CHUNK)
    w = edge_weight.reshape(NC, NS, NCHUNK, CHUNK)
    pad = ((0, NPAD - N), (0, 0))
    ah0p = jnp.pad(ah0, pad)
    init = jnp.stack([ah0p, jnp.zeros_like(ah0p)])
    prop = _make_prop()
    h = jnp.pad(h0, pad)
    for _ in range(K):
        partials = prop(h, src, dst, w, init)
        h = _combine(partials)
    return h[:N]


# speedup vs baseline: 3.0247x; 1.0315x over previous
import functools

import jax
import jax.numpy as jnp
from jax import lax
from jax.experimental import pallas as pl
from jax.experimental.pallas import tpu as pltpu
from jax.experimental.pallas import tpu_sc as plsc

N = 10000
E = 320000
D = 128
H = 128
C = 64
K = 10
ALPHA = 0.1

NC = 2
NS = 16
LANES = 16
EDGES_PER_TILE = E // (NC * NS)
CHUNK = 80
NCHUNK = EDGES_PER_TILE // CHUNK
NPAD = 10240
ROWS_PER_TILE = NPAD // NS

ROW_BLK = 2000
CMB_BLK = 2048


def _mlp_body(x_ref, w1_ref, b1_ref, w2_ref, b2_ref, h_ref, ah_ref):
    h1 = jnp.maximum(
        jnp.dot(x_ref[...], w1_ref[...], preferred_element_type=jnp.float32)
        + b1_ref[...], 0.0)
    h2 = (jnp.dot(h1, w2_ref[...], preferred_element_type=jnp.float32)
          + b2_ref[...])
    h_ref[...] = h2
    ah_ref[...] = ALPHA * h2


def _mlp(features, W1, b1, W2, b2):
    grid = (N // ROW_BLK,)
    return pl.pallas_call(
        _mlp_body,
        grid=grid,
        in_specs=[
            pl.BlockSpec((ROW_BLK, D), lambda i: (i, 0)),
            pl.BlockSpec((D, H), lambda i: (0, 0)),
            pl.BlockSpec((1, H), lambda i: (0, 0)),
            pl.BlockSpec((H, C), lambda i: (0, 0)),
            pl.BlockSpec((1, C), lambda i: (0, 0)),
        ],
        out_specs=[
            pl.BlockSpec((ROW_BLK, C), lambda i: (i, 0)),
            pl.BlockSpec((ROW_BLK, C), lambda i: (i, 0)),
        ],
        out_shape=[
            jax.ShapeDtypeStruct((N, C), jnp.float32),
            jax.ShapeDtypeStruct((N, C), jnp.float32),
        ],
    )(features, W1, b1.reshape(1, H), W2, b2.reshape(1, C))


def _combine_body(p_ref, o_ref):
    o_ref[...] = p_ref[0] + p_ref[1]


def _combine(partials):
    grid = (NPAD // CMB_BLK,)
    return pl.pallas_call(
        _combine_body,
        grid=grid,
        in_specs=[pl.BlockSpec((NC, CMB_BLK, C), lambda i: (0, i, 0))],
        out_specs=pl.BlockSpec((CMB_BLK, C), lambda i: (i, 0)),
        out_shape=jax.ShapeDtypeStruct((NPAD, C), jnp.float32),
    )(partials)


_GATHER_DNUMS = lax.GatherDimensionNumbers(
    offset_dims=(), collapsed_slice_dims=(0,), start_index_map=(0,))


def _bcast_lane(vec, lane):
    idx = jnp.full((LANES, 1), lane, jnp.int32)
    return lax.gather(vec, idx, _GATHER_DNUMS, slice_sizes=(1,),
                      mode=lax.GatherScatterMode.PROMISE_IN_BOUNDS)


NBUF = 5


def _prop_body(h_hbm, src_hbm, dst_hbm, w_hbm, init_hbm, out_hbm,
               src_v, dst_v, w_v, rin_v, rout_v, agg_sh, gsem, ssem):
    c = lax.axis_index("c")
    s = lax.axis_index("s")

    rows_sl = pl.ds(s * ROWS_PER_TILE, ROWS_PER_TILE)
    pltpu.async_copy(src_hbm.at[c, s], src_v, ssem.at[0])
    pltpu.async_copy(dst_hbm.at[c, s], dst_v, ssem.at[1])
    pltpu.async_copy(w_hbm.at[c, s], w_v, ssem.at[2])
    pltpu.async_copy(init_hbm.at[c].at[rows_sl], agg_sh.at[rows_sl],
                     ssem.at[3])
    pltpu.make_async_copy(src_hbm.at[c, s], src_v, ssem.at[0]).wait()
    pltpu.make_async_copy(dst_hbm.at[c, s], dst_v, ssem.at[1]).wait()
    pltpu.make_async_copy(w_hbm.at[c, s], w_v, ssem.at[2]).wait()
    pltpu.make_async_copy(init_hbm.at[c].at[rows_sl], agg_sh.at[rows_sl],
                          ssem.at[3]).wait()
    plsc.subcore_barrier()

    for b in range(NBUF):
        pltpu.async_copy(h_hbm.at[src_v.at[b]], rin_v.at[b], gsem.at[b])

    @pl.loop(0, NCHUNK, step=NBUF)
    def _(g0):
        for b in range(NBUF):
            j = g0 + b
            pltpu.make_async_copy(h_hbm.at[src_v.at[j]], rin_v.at[b],
                                  gsem.at[b]).wait()
            @pl.when(g0 > 0)
            def _():
                pltpu.make_async_copy(rout_v.at[b], agg_sh.at[dst_v.at[j]],
                                      ssem.at[b]).wait()
            @pl.loop(0, CHUNK // LANES)
            def _(e5):
                w16 = w_v[j, pl.ds(e5 * LANES, LANES)] * (1.0 - ALPHA)
                for e in range(LANES):
                    wb = _bcast_lane(w16, e)
                    row = e5 * LANES + e
                    for f in range(C // LANES):
                        sl = (pl.ds(row, 1), pl.ds(f * LANES, LANES))
                        rout_v[b, row, pl.ds(f * LANES, LANES)] = (
                            rin_v[b, row, pl.ds(f * LANES, LANES)] * wb)
            @pl.when(j + NBUF < NCHUNK)
            def _():
                pltpu.async_copy(h_hbm.at[src_v.at[j + NBUF]], rin_v.at[b],
                                 gsem.at[b])
            pltpu.async_copy(rout_v.at[b], agg_sh.at[dst_v.at[j]],
                             ssem.at[b], add=True)

    for b in range(NBUF):
        pltpu.make_async_copy(rout_v.at[b],
                              agg_sh.at[dst_v.at[NCHUNK - NBUF + b]],
                              ssem.at[b]).wait()

    plsc.subcore_barrier()
    pltpu.sync_copy(agg_sh.at[pl.ds(s * ROWS_PER_TILE, ROWS_PER_TILE)],
                    out_hbm.at[c].at[pl.ds(s * ROWS_PER_TILE, ROWS_PER_TILE)])


def _make_prop():
    mesh = plsc.VectorSubcoreMesh(core_axis_name="c", subcore_axis_name="s")
    return pl.kernel(
        _prop_body,
        mesh=mesh,
        out_type=jax.ShapeDtypeStruct((NC, NPAD, C), jnp.float32),
        scratch_types=[
            pltpu.VMEM((NCHUNK, CHUNK), jnp.int32),
            pltpu.VMEM((NCHUNK, CHUNK), jnp.int32),
            pltpu.VMEM((NCHUNK, CHUNK), jnp.float32),
            pltpu.VMEM((NBUF, CHUNK, C), jnp.float32),
            pltpu.VMEM((NBUF, CHUNK, C), jnp.float32),
            pltpu.VMEM_SHARED((NPAD, C), jnp.float32),
            pltpu.SemaphoreType.DMA((NBUF,)),
            pltpu.SemaphoreType.DMA((NBUF,)),
        ],
        compiler_params=pltpu.CompilerParams(use_tc_tiling_on_sc=False),
    )


def kernel(features, edge_weight, edge_index, W1, b1, W2, b2):
    h0, ah0 = _mlp(features, W1, b1, W2, b2)
    src = edge_index[0].reshape(NC, NS, NCHUNK, CHUNK)
    dst = edge_index[1].reshape(NC, NS, NCHUNK, CHUNK)
    w = edge_weight.reshape(NC, NS, NCHUNK, CHUNK)
    pad = ((0, NPAD - N), (0, 0))
    ah0p = jnp.pad(ah0, pad)
    init = jnp.stack([ah0p, jnp.zeros_like(ah0p)])
    prop = _make_prop()
    h = jnp.pad(h0, pad)
    for _ in range(K):
        partials = prop(h, src, dst, w, init)
        h = _combine(partials)
    return h[:N]


# speedup vs baseline: 3.0263x; 1.0005x over previous
import jax
import jax.numpy as jnp
from jax import lax
from jax.experimental import pallas as pl
from jax.experimental.pallas import tpu as pltpu
from jax.experimental.pallas import tpu_sc as plsc

N = 10000
E = 320000
D = 128
H = 128
C = 64
K = 10
ALPHA = 0.1

NC = 2
NS = 16
LANES = 16
EDGES_PER_TILE = E // (NC * NS)
CHUNK = 80
NCHUNK = EDGES_PER_TILE // CHUNK
NPAD = 10240
ROWS_PER_TILE = NPAD // NS

ROW_BLK = 2000
CMB_BLK = 2048


def _mlp_body(x_ref, w1_ref, b1_ref, w2_ref, b2_ref, h_ref, ah_ref):
    h1 = jnp.maximum(
        jnp.dot(x_ref[...], w1_ref[...], preferred_element_type=jnp.float32)
        + b1_ref[...], 0.0)
    h2 = (jnp.dot(h1, w2_ref[...], preferred_element_type=jnp.float32)
          + b2_ref[...])
    h_ref[...] = h2
    ah_ref[...] = ALPHA * h2


def _mlp(features, W1, b1, W2, b2):
    grid = (N // ROW_BLK,)
    return pl.pallas_call(
        _mlp_body,
        grid=grid,
        in_specs=[
            pl.BlockSpec((ROW_BLK, D), lambda i: (i, 0)),
            pl.BlockSpec((D, H), lambda i: (0, 0)),
            pl.BlockSpec((1, H), lambda i: (0, 0)),
            pl.BlockSpec((H, C), lambda i: (0, 0)),
            pl.BlockSpec((1, C), lambda i: (0, 0)),
        ],
        out_specs=[
            pl.BlockSpec((ROW_BLK, C), lambda i: (i, 0)),
            pl.BlockSpec((ROW_BLK, C), lambda i: (i, 0)),
        ],
        out_shape=[
            jax.ShapeDtypeStruct((N, C), jnp.float32),
            jax.ShapeDtypeStruct((N, C), jnp.float32),
        ],
    )(features, W1, b1.reshape(1, H), W2, b2.reshape(1, C))


def _combine_body(p_ref, o_ref):
    o_ref[...] = p_ref[0] + p_ref[1]


def _combine(partials):
    grid = (NPAD // CMB_BLK,)
    return pl.pallas_call(
        _combine_body,
        grid=grid,
        in_specs=[pl.BlockSpec((NC, CMB_BLK, C), lambda i: (0, i, 0))],
        out_specs=pl.BlockSpec((CMB_BLK, C), lambda i: (i, 0)),
        out_shape=jax.ShapeDtypeStruct((NPAD, C), jnp.float32),
    )(partials)


_GATHER_DNUMS = lax.GatherDimensionNumbers(
    offset_dims=(), collapsed_slice_dims=(0,), start_index_map=(0,))


def _bcast_lane(vec, lane):
    idx = jnp.full((LANES, 1), lane, jnp.int32)
    return lax.gather(vec, idx, _GATHER_DNUMS, slice_sizes=(1,),
                      mode=lax.GatherScatterMode.PROMISE_IN_BOUNDS)


NBUF = 5


def _prop_body(h_hbm, src_hbm, dst_hbm, w_hbm, init_hbm, out_hbm,
               src_v, dst_v, w_v, rin_v, rout_v, agg_sh, gsem, ssem):
    c = lax.axis_index("c")
    s = lax.axis_index("s")

    rows_sl = pl.ds(s * ROWS_PER_TILE, ROWS_PER_TILE)
    pltpu.async_copy(src_hbm.at[c, s], src_v, ssem.at[0])
    pltpu.async_copy(dst_hbm.at[c, s], dst_v, ssem.at[1])
    pltpu.async_copy(w_hbm.at[c, s], w_v, ssem.at[2])
    pltpu.async_copy(init_hbm.at[c].at[rows_sl], agg_sh.at[rows_sl],
                     ssem.at[3])
    pltpu.make_async_copy(src_hbm.at[c, s], src_v, ssem.at[0]).wait()
    pltpu.make_async_copy(dst_hbm.at[c, s], dst_v, ssem.at[1]).wait()
    pltpu.make_async_copy(w_hbm.at[c, s], w_v, ssem.at[2]).wait()
    pltpu.make_async_copy(init_hbm.at[c].at[rows_sl], agg_sh.at[rows_sl],
                          ssem.at[3]).wait()
    plsc.subcore_barrier()

    for b in range(NBUF):
        pltpu.async_copy(h_hbm.at[src_v.at[b]], rin_v.at[b], gsem.at[b])

    @pl.loop(0, NCHUNK, step=NBUF)
    def _(g0):
        for b in range(NBUF):
            j = g0 + b
            pltpu.make_async_copy(h_hbm.at[src_v.at[j]], rin_v.at[b],
                                  gsem.at[b]).wait()
            @pl.when(g0 > 0)
            def _():
                pltpu.make_async_copy(rout_v.at[b], agg_sh.at[dst_v.at[j]],
                                      ssem.at[b]).wait()
            @pl.loop(0, CHUNK // LANES)
            def _(e5):
                w16 = w_v[j, pl.ds(e5 * LANES, LANES)] * (1.0 - ALPHA)
                for e in range(LANES):
                    wb = _bcast_lane(w16, e)
                    row = e5 * LANES + e
                    for f in range(C // LANES):
                        rout_v[b, row, pl.ds(f * LANES, LANES)] = (
                            rin_v[b, row, pl.ds(f * LANES, LANES)] * wb)
            @pl.when(j + NBUF < NCHUNK)
            def _():
                pltpu.async_copy(h_hbm.at[src_v.at[j + NBUF]], rin_v.at[b],
                                 gsem.at[b])
            pltpu.async_copy(rout_v.at[b], agg_sh.at[dst_v.at[j]],
                             ssem.at[b], add=True)

    for b in range(NBUF):
        pltpu.make_async_copy(rout_v.at[b],
                              agg_sh.at[dst_v.at[NCHUNK - NBUF + b]],
                              ssem.at[b]).wait()

    plsc.subcore_barrier()
    pltpu.sync_copy(agg_sh.at[pl.ds(s * ROWS_PER_TILE, ROWS_PER_TILE)],
                    out_hbm.at[c].at[pl.ds(s * ROWS_PER_TILE, ROWS_PER_TILE)])


def _make_prop():
    mesh = plsc.VectorSubcoreMesh(core_axis_name="c", subcore_axis_name="s")
    return pl.kernel(
        _prop_body,
        mesh=mesh,
        out_type=jax.ShapeDtypeStruct((NC, NPAD, C), jnp.float32),
        scratch_types=[
            pltpu.VMEM((NCHUNK, CHUNK), jnp.int32),
            pltpu.VMEM((NCHUNK, CHUNK), jnp.int32),
            pltpu.VMEM((NCHUNK, CHUNK), jnp.float32),
            pltpu.VMEM((NBUF, CHUNK, C), jnp.float32),
            pltpu.VMEM((NBUF, CHUNK, C), jnp.float32),
            pltpu.VMEM_SHARED((NPAD, C), jnp.float32),
            pltpu.SemaphoreType.DMA((NBUF,)),
            pltpu.SemaphoreType.DMA((NBUF,)),
        ],
        compiler_params=pltpu.CompilerParams(use_tc_tiling_on_sc=False),
    )


def kernel(features, edge_weight, edge_index, W1, b1, W2, b2):
    h0, ah0 = _mlp(features, W1, b1, W2, b2)
    src = edge_index[0].reshape(NC, NS, NCHUNK, CHUNK)
    dst = edge_index[1].reshape(NC, NS, NCHUNK, CHUNK)
    w = edge_weight.reshape(NC, NS, NCHUNK, CHUNK)
    pad = ((0, NPAD - N), (0, 0))
    ah0p = jnp.pad(ah0, pad)
    init = jnp.stack([ah0p, jnp.zeros_like(ah0p)])
    prop = _make_prop()
    h = jnp.pad(h0, pad)
    for _ in range(K):
        partials = prop(h, src, dst, w, init)
        h = _combine(partials)
    return h[:N]


# speedup vs baseline: 3.5314x; 1.1669x over previous
import jax
import jax.numpy as jnp
from jax import lax
from jax.experimental import pallas as pl
from jax.experimental.pallas import tpu as pltpu
from jax.experimental.pallas import tpu_sc as plsc

N = 10000
E = 320000
D = 128
H = 128
C = 64
K = 10
ALPHA = 0.1

NC = 2
NS = 16
LANES = 16
EDGES_PER_TILE = E // (NC * NS)
CHUNK = 80
NCHUNK = EDGES_PER_TILE // CHUNK
NPAD = 10240
ROWS_PER_TILE = NPAD // NS

ROW_BLK = 2000
CMB_BLK = 2048


def _mlp_body(x_ref, w1_ref, b1_ref, w2_ref, b2_ref, h_ref, ah_ref):
    h1 = jnp.maximum(
        jnp.dot(x_ref[...], w1_ref[...], preferred_element_type=jnp.float32)
        + b1_ref[...], 0.0)
    h2 = (jnp.dot(h1, w2_ref[...], preferred_element_type=jnp.float32)
          + b2_ref[...])
    h_ref[...] = h2
    ah_ref[...] = ALPHA * h2


def _mlp(features, W1, b1, W2, b2):
    grid = (N // ROW_BLK,)
    return pl.pallas_call(
        _mlp_body,
        grid=grid,
        in_specs=[
            pl.BlockSpec((ROW_BLK, D), lambda i: (i, 0)),
            pl.BlockSpec((D, H), lambda i: (0, 0)),
            pl.BlockSpec((1, H), lambda i: (0, 0)),
            pl.BlockSpec((H, C), lambda i: (0, 0)),
            pl.BlockSpec((1, C), lambda i: (0, 0)),
        ],
        out_specs=[
            pl.BlockSpec((ROW_BLK, C), lambda i: (i, 0)),
            pl.BlockSpec((ROW_BLK, C), lambda i: (i, 0)),
        ],
        out_shape=[
            jax.ShapeDtypeStruct((N, C), jnp.float32),
            jax.ShapeDtypeStruct((N, C), jnp.float32),
        ],
    )(features, W1, b1.reshape(1, H), W2, b2.reshape(1, C))


def _combine_body(p_ref, o_ref):
    o_ref[...] = p_ref[0] + p_ref[1]


def _combine(partials):
    grid = (NPAD // CMB_BLK,)
    return pl.pallas_call(
        _combine_body,
        grid=grid,
        in_specs=[pl.BlockSpec((NC, CMB_BLK, C), lambda i: (0, i, 0))],
        out_specs=pl.BlockSpec((CMB_BLK, C), lambda i: (i, 0)),
        out_shape=jax.ShapeDtypeStruct((NPAD, C), jnp.float32),
    )(partials)


_GATHER_DNUMS = lax.GatherDimensionNumbers(
    offset_dims=(), collapsed_slice_dims=(0,), start_index_map=(0,))


def _bcast_lane(vec, lane):
    idx = jnp.full((LANES, 1), lane, jnp.int32)
    return lax.gather(vec, idx, _GATHER_DNUMS, slice_sizes=(1,),
                      mode=lax.GatherScatterMode.PROMISE_IN_BOUNDS)


NBUF = 5


CROWS = NPAD // (NC * NS)
CSUB = CROWS // CHUNK


def _prop_body(p_hbm, src_hbm, dst_hbm, w_hbm, init_hbm, h_hbm, out_hbm,
               src_v, dst_v, w_v, rin_v, rout_v, agg_sh, gsem, ssem, psem,
               xsem):
    c = lax.axis_index("c")
    s = lax.axis_index("s")
    t = c * NS + s

    rows_sl = pl.ds(s * ROWS_PER_TILE, ROWS_PER_TILE)
    pltpu.async_copy(src_hbm.at[c, s], src_v, ssem.at[0])
    pltpu.async_copy(dst_hbm.at[c, s], dst_v, ssem.at[1])
    pltpu.async_copy(w_hbm.at[c, s], w_v, ssem.at[2])
    pltpu.async_copy(init_hbm.at[c].at[rows_sl], agg_sh.at[rows_sl],
                     ssem.at[3])

    for i in range(CSUB):
        csl = pl.ds(t * CROWS + i * CHUNK, CHUNK)
        pltpu.async_copy(p_hbm.at[0].at[csl], rin_v.at[i], gsem.at[i])
        pltpu.async_copy(p_hbm.at[1].at[csl], rout_v.at[i], psem.at[i])
    for i in range(CSUB):
        csl = pl.ds(t * CROWS + i * CHUNK, CHUNK)
        pltpu.make_async_copy(p_hbm.at[0].at[csl], rin_v.at[i],
                              gsem.at[i]).wait()
        pltpu.make_async_copy(p_hbm.at[1].at[csl], rout_v.at[i],
                              psem.at[i]).wait()

        @pl.loop(0, CHUNK)
        def _(r):
            for f in range(C // LANES):
                fsl = pl.ds(f * LANES, LANES)
                rin_v[i, r, fsl] = rin_v[i, r, fsl] + rout_v[i, r, fsl]
        pltpu.async_copy(rin_v.at[i], h_hbm.at[csl], gsem.at[i])
    for i in range(CSUB):
        csl = pl.ds(t * CROWS + i * CHUNK, CHUNK)
        pltpu.make_async_copy(rin_v.at[i], h_hbm.at[csl], gsem.at[i]).wait()
    pltpu.make_async_copy(src_hbm.at[c, s], src_v, ssem.at[0]).wait()
    pltpu.make_async_copy(dst_hbm.at[c, s], dst_v, ssem.at[1]).wait()
    pltpu.make_async_copy(w_hbm.at[c, s], w_v, ssem.at[2]).wait()
    pltpu.make_async_copy(init_hbm.at[c].at[rows_sl], agg_sh.at[rows_sl],
                          ssem.at[3]).wait()

    plsc.subcore_barrier()

    @pl.when(s == 0)
    def _():
        pl.semaphore_signal(xsem, 1, core_index=1 - c)
        pl.semaphore_wait(xsem, 1)
    plsc.subcore_barrier()

    for b in range(NBUF):
        pltpu.async_copy(h_hbm.at[src_v.at[b]], rin_v.at[b], gsem.at[b])

    @pl.loop(0, NCHUNK, step=NBUF)
    def _(g0):
        for b in range(NBUF):
            j = g0 + b
            pltpu.make_async_copy(h_hbm.at[src_v.at[j]], rin_v.at[b],
                                  gsem.at[b]).wait()
            @pl.when(g0 > 0)
            def _():
                pltpu.make_async_copy(rout_v.at[b], agg_sh.at[dst_v.at[j]],
                                      ssem.at[b]).wait()
            @pl.loop(0, CHUNK // LANES)
            def _(e5):
                w16 = w_v[j, pl.ds(e5 * LANES, LANES)] * (1.0 - ALPHA)
                for e in range(LANES):
                    wb = _bcast_lane(w16, e)
                    row = e5 * LANES + e
                    for f in range(C // LANES):
                        rout_v[b, row, pl.ds(f * LANES, LANES)] = (
                            rin_v[b, row, pl.ds(f * LANES, LANES)] * wb)
            @pl.when(j + NBUF < NCHUNK)
            def _():
                pltpu.async_copy(h_hbm.at[src_v.at[j + NBUF]], rin_v.at[b],
                                 gsem.at[b])
            pltpu.async_copy(rout_v.at[b], agg_sh.at[dst_v.at[j]],
                             ssem.at[b], add=True)

    for b in range(NBUF):
        pltpu.make_async_copy(rout_v.at[b],
                              agg_sh.at[dst_v.at[NCHUNK - NBUF + b]],
                              ssem.at[b]).wait()

    plsc.subcore_barrier()
    pltpu.sync_copy(agg_sh.at[pl.ds(s * ROWS_PER_TILE, ROWS_PER_TILE)],
                    out_hbm.at[c].at[pl.ds(s * ROWS_PER_TILE, ROWS_PER_TILE)])


def _make_prop():
    mesh = plsc.VectorSubcoreMesh(core_axis_name="c", subcore_axis_name="s")
    return pl.kernel(
        _prop_body,
        mesh=mesh,
        out_type=[
            jax.ShapeDtypeStruct((NPAD, C), jnp.float32),
            jax.ShapeDtypeStruct((NC, NPAD, C), jnp.float32),
        ],
        scratch_types=[
            pltpu.VMEM((NCHUNK, CHUNK), jnp.int32),
            pltpu.VMEM((NCHUNK, CHUNK), jnp.int32),
            pltpu.VMEM((NCHUNK, CHUNK), jnp.float32),
            pltpu.VMEM((NBUF, CHUNK, C), jnp.float32),
            pltpu.VMEM((NBUF, CHUNK, C), jnp.float32),
            pltpu.VMEM_SHARED((NPAD, C), jnp.float32),
            pltpu.SemaphoreType.DMA((NBUF,)),
            pltpu.SemaphoreType.DMA((NBUF,)),
            pltpu.SemaphoreType.DMA((CSUB,)),
            pltpu.SemaphoreType.REGULAR,
        ],
        compiler_params=pltpu.CompilerParams(use_tc_tiling_on_sc=False),
    )


def kernel(features, edge_weight, edge_index, W1, b1, W2, b2):
    h0, ah0 = _mlp(features, W1, b1, W2, b2)
    src = edge_index[0].reshape(NC, NS, NCHUNK, CHUNK)
    dst = edge_index[1].reshape(NC, NS, NCHUNK, CHUNK)
    w = edge_weight.reshape(NC, NS, NCHUNK, CHUNK)
    pad = ((0, NPAD - N), (0, 0))
    ah0p = jnp.pad(ah0, pad)
    init = jnp.stack([ah0p, jnp.zeros_like(ah0p)])
    prop = _make_prop()
    partials = jnp.stack([jnp.pad(h0, pad), jnp.zeros_like(ah0p)])
    for _ in range(K):
        _h, partials = prop(partials, src, dst, w, init)
    return _combine(partials)[:N]
